# baseline (speedup 1.0000x reference)
import jax
import jax.numpy as jnp
from jax.experimental import pallas as pl
from jax.experimental.pallas import tpu as pltpu


def _halfblock(x, tgt, lens, t0):
    Th, Bb, Vb = x.shape
    m = jnp.max(x, axis=-1)
    idx = jax.lax.broadcasted_iota(jnp.int32, x.shape, 2)
    cand = jnp.where(x == m[..., None], idx, Vb)
    pred = jnp.min(cand, axis=-1)
    tids = t0 + jax.lax.broadcasted_iota(jnp.int32, (Th, Bb), 0)
    mask = tids < (lens - 1)
    corr = jnp.logical_and(pred == tgt, mask)
    c = jnp.sum(corr.astype(jnp.float32))
    v = jnp.sum(mask.astype(jnp.float32))
    return c, v


def _partial_body(lens_ref, x1_ref, x2_ref, tgt_ref, part_ref):
    i = pl.program_id(0)
    Th = x1_ref.shape[0]
    lens = lens_ref[...]
    tgt = tgt_ref[...]
    c1, v1 = _halfblock(x1_ref[...], tgt[:Th], lens, i * 2 * Th)
    c2, v2 = _halfblock(x2_ref[...], tgt[Th:], lens, i * 2 * Th + Th)
    c = c1 + c2
    v = v1 + v2
    lane = jax.lax.broadcasted_iota(jnp.int32, (1, 128), 1)
    row = jnp.where(lane == 0, c, jnp.where(lane == 1, v, 0.0))
    part_ref[...] = row.reshape(1, 1, 128)


def _finish_body(part_ref, out_ref):
    p = part_ref[...].reshape(part_ref.shape[0], 128)
    lane = jax.lax.broadcasted_iota(jnp.int32, p.shape, 1)
    c = jnp.sum(jnp.where(lane == 0, p, 0.0))
    v = jnp.sum(jnp.where(lane == 1, p, 0.0))
    out_ref[...] = jnp.full((1, 128), c / v, dtype=jnp.float32)


def kernel(outputs, tokens, tokens_lens):
    T, B, V = outputs.shape
    Tb = 64
    Th = Tb // 2
    n = T // Tb
    targets = jnp.roll(tokens, -1, axis=0)
    lens2d = tokens_lens.reshape(1, B)

    parts = pl.pallas_call(
        _partial_body,
        grid=(n,),
        in_specs=[
            pl.BlockSpec((1, B), lambda i: (0, 0)),
            pl.BlockSpec((Th, B, V), lambda i: (2 * i, 0, 0)),
            pl.BlockSpec((Th, B, V), lambda i: (2 * i + 1, 0, 0)),
            pl.BlockSpec((Tb, B), lambda i: (i, 0)),
        ],
        out_specs=pl.BlockSpec((1, 1, 128), lambda i: (i, 0, 0)),
        out_shape=jax.ShapeDtypeStruct((n, 1, 128), jnp.float32),
        compiler_params=pltpu.CompilerParams(
            dimension_semantics=("parallel",),
        ),
    )(lens2d, outputs, outputs, targets)

    acc = pl.pallas_call(
        _finish_body,
        out_shape=jax.ShapeDtypeStruct((1, 128), jnp.float32),
    )(parts)
    return acc[0, 0]


# speedup vs baseline: 1.0207x; 1.0207x over previous
import jax
import jax.numpy as jnp
from jax.experimental import pallas as pl
from jax.experimental.pallas import tpu as pltpu


def _halfblock(x, tgt, lens, t0):
    Th, Bb, Vb = x.shape
    m = jnp.max(x, axis=-1)
    idx = jax.lax.broadcasted_iota(jnp.int32, x.shape, 2)
    cand = jnp.where(x == m[..., None], idx, Vb)
    pred = jnp.min(cand, axis=-1)
    tids = t0 + jax.lax.broadcasted_iota(jnp.int32, (Th, Bb), 0)
    mask = tids < (lens - 1)
    corr = jnp.logical_and(pred == tgt, mask)
    c = jnp.sum(corr.astype(jnp.float32))
    v = jnp.sum(mask.astype(jnp.float32))
    return c, v


def _body(lens_ref, x1_ref, x2_ref, tgt_ref, out_ref, acc_ref):
    i = pl.program_id(0)

    @pl.when(i == 0)
    def _init():
        acc_ref[0] = 0.0
        acc_ref[1] = 0.0

    Th = x1_ref.shape[0]
    lens = lens_ref[...]
    tgt = tgt_ref[...]
    c1, v1 = _halfblock(x1_ref[...], tgt[:Th], lens, i * 2 * Th)
    c2, v2 = _halfblock(x2_ref[...], tgt[Th:], lens, i * 2 * Th + Th)
    acc_ref[0] += c1 + c2
    acc_ref[1] += v1 + v2

    @pl.when(i == pl.num_programs(0) - 1)
    def _fini():
        out_ref[...] = jnp.full((1, 128), acc_ref[0] / acc_ref[1],
                                dtype=jnp.float32)


def kernel(outputs, tokens, tokens_lens):
    T, B, V = outputs.shape
    Tb = 64
    Th = Tb // 2
    n = T // Tb
    targets = jnp.roll(tokens, -1, axis=0)
    lens2d = tokens_lens.reshape(1, B)

    acc = pl.pallas_call(
        _body,
        grid=(n,),
        in_specs=[
            pl.BlockSpec((1, B), lambda i: (0, 0)),
            pl.BlockSpec((Th, B, V), lambda i: (2 * i, 0, 0)),
            pl.BlockSpec((Th, B, V), lambda i: (2 * i + 1, 0, 0)),
            pl.BlockSpec((Tb, B), lambda i: (i, 0)),
        ],
        out_specs=pl.BlockSpec((1, 128), lambda i: (0, 0)),
        out_shape=jax.ShapeDtypeStruct((1, 128), jnp.float32),
        scratch_shapes=[pltpu.SMEM((2,), jnp.float32)],
        compiler_params=pltpu.CompilerParams(
            dimension_semantics=("arbitrary",),
        ),
    )(lens2d, outputs, outputs, targets)
    return acc[0, 0]
